# Initial kernel scaffold; baseline (speedup 1.0000x reference)
import functools

import jax
import jax.numpy as jnp
from jax import lax
from jax.experimental import pallas as pl
from jax.experimental.pallas import tpu as pltpu
from jax.experimental.pallas import tpu_sc as plsc

N = 10000
E = 320000
D = 128
H = 128

NPAD = 10240
NCORES = 2
NSUB = 16
NW = NCORES * NSUB
CH = 128
EPT = 10112
NITER = EPT // CH
EPAD = NW * EPT
RPT = NPAD // NSUB

_mesh = plsc.VectorSubcoreMesh(core_axis_name="c", subcore_axis_name="s")


@functools.partial(
    pl.kernel,
    out_type=jax.ShapeDtypeStruct((NCORES, NPAD, 16), jnp.float32),
    mesh=_mesh,
    scratch_types=[
        pltpu.VMEM((CH,), jnp.int32),
        pltpu.VMEM((CH, 16), jnp.float32),
        pltpu.VMEM((RPT, 16), jnp.float32),
        pltpu.VMEM_SHARED((NPAD, 16), jnp.float32),
    ],
)
def _deg_kernel(dst_hbm, out_hbm, idx_d, ones_v, zeros_v, dacc):
    c = lax.axis_index("c")
    s = lax.axis_index("s")
    tile = c * NSUB + s
    r0 = s * RPT
    one16 = jnp.full((16,), 1.0, jnp.float32)
    zero16 = jnp.zeros((16,), jnp.float32)

    def fill_ones(i, _):
        ones_v[i, :] = one16
        return 0

    lax.fori_loop(0, CH, fill_ones, 0)

    def fill_zeros(i, _):
        zeros_v[i, :] = zero16
        return 0

    lax.fori_loop(0, RPT, fill_zeros, 0)
    pltpu.sync_copy(zeros_v, dacc.at[pl.ds(r0, RPT)])
    plsc.subcore_barrier()

    def body(i, _):
        b = tile * EPT + i * CH
        pltpu.sync_copy(dst_hbm.at[pl.ds(b, CH)], idx_d)
        pltpu.sync_copy(ones_v, dacc.at[idx_d], add=True)
        return 0

    lax.fori_loop(0, NITER, body, 0)
    plsc.subcore_barrier()
    pltpu.sync_copy(dacc.at[pl.ds(r0, RPT)], out_hbm.at[c, pl.ds(r0, RPT)])


@functools.partial(
    pl.kernel,
    out_type=jax.ShapeDtypeStruct((NCORES, NPAD, H), jnp.float32),
    mesh=_mesh,
    scratch_types=[
        pltpu.VMEM((CH,), jnp.int32),
        pltpu.VMEM((CH,), jnp.int32),
        pltpu.VMEM((CH, H), jnp.float32),
        pltpu.VMEM_SHARED((NPAD, H), jnp.float32),
        pltpu.SemaphoreType.DMA,
    ],
)
def _scatter_kernel(y_hbm, src_hbm, dst_hbm, out_hbm, idx_s, idx_d, rows, acc, sem):
    c = lax.axis_index("c")
    s = lax.axis_index("s")
    tile = c * NSUB + s
    r0 = s * RPT
    pltpu.sync_copy(y_hbm.at[pl.ds(r0, RPT)], acc.at[pl.ds(r0, RPT)])
    plsc.subcore_barrier()

    def body(i, _):
        b = tile * EPT + i * CH
        pltpu.sync_copy(src_hbm.at[pl.ds(b, CH)], idx_s)
        pltpu.sync_copy(dst_hbm.at[pl.ds(b, CH)], idx_d)
        pltpu.async_copy(y_hbm.at[idx_s], rows, sem).wait()
        pltpu.sync_copy(rows, acc.at[idx_d], add=True)
        return 0

    lax.fori_loop(0, NITER, body, 0)
    plsc.subcore_barrier()
    pltpu.sync_copy(acc.at[pl.ds(r0, RPT)], out_hbm.at[c, pl.ds(r0, RPT)])


def _dinv(d0_ref, d1_ref):
    deg = d0_ref[:, 0:1] + d1_ref[:, 0:1] + 1.0
    return lax.rsqrt(deg)


def _mm_scale_body(x_ref, w_ref, d0_ref, d1_ref, o_ref):
    y = jnp.dot(x_ref[...], w_ref[...], preferred_element_type=jnp.float32)
    o_ref[...] = y * _dinv(d0_ref, d1_ref)


def _mid_body(p0_ref, p1_ref, y_ref, d0_ref, d1_ref, b_ref, w_ref, o_ref):
    dinv = _dinv(d0_ref, d1_ref)
    h = (p0_ref[...] + p1_ref[...] - y_ref[...]) * dinv + b_ref[...]
    h = jnp.maximum(h, 0.0)
    o_ref[...] = jnp.dot(h, w_ref[...], preferred_element_type=jnp.float32) * dinv


def _final_body(q0_ref, q1_ref, y_ref, d0_ref, d1_ref, b_ref, o_ref):
    dinv = _dinv(d0_ref, d1_ref)
    o_ref[...] = (q0_ref[...] + q1_ref[...] - y_ref[...]) * dinv + b_ref[...]


_BM = 2048
_row = pl.BlockSpec((_BM, H), lambda i: (i, 0))
_deg = pl.BlockSpec((_BM, 16), lambda i: (i, 0))
_w = pl.BlockSpec((D, H), lambda i: (0, 0))
_b = pl.BlockSpec((1, H), lambda i: (0, 0))
_grid = (NPAD // _BM,)

_mm_scale = pl.pallas_call(
    _mm_scale_body,
    grid=_grid,
    in_specs=[_row, _w, _deg, _deg],
    out_specs=_row,
    out_shape=jax.ShapeDtypeStruct((NPAD, H), jnp.float32),
)

_mid = pl.pallas_call(
    _mid_body,
    grid=_grid,
    in_specs=[_row, _row, _row, _deg, _deg, _b, _w],
    out_specs=_row,
    out_shape=jax.ShapeDtypeStruct((NPAD, H), jnp.float32),
)

_final = pl.pallas_call(
    _final_body,
    grid=_grid,
    in_specs=[_row, _row, _row, _deg, _deg, _b],
    out_specs=_row,
    out_shape=jax.ShapeDtypeStruct((NPAD, H), jnp.float32),
)


def kernel(x, W1, b1, W2, b2, edge_index, scope):
    del scope
    xp = jnp.zeros((NPAD, D), jnp.float32).at[:N].set(x)
    pad = EPAD - E
    srcp = jnp.concatenate([edge_index[0], jnp.zeros((pad,), jnp.int32)])
    dstp = jnp.concatenate([edge_index[1], jnp.full((pad,), N, jnp.int32)])

    degp = _deg_kernel(dstp)
    deg0, deg1 = degp[0], degp[1]
    b1r = b1.reshape(1, H)
    b2r = b2.reshape(1, H)

    y1 = _mm_scale(xp, W1, deg0, deg1)
    p = _scatter_kernel(y1, srcp, dstp)
    y2 = _mid(p[0], p[1], y1, deg0, deg1, b1r, W2)
    q = _scatter_kernel(y2, srcp, dstp)
    out = _final(q[0], q[1], y2, deg0, deg1, b2r)
    return out[:N]


# speedup vs baseline: 10.4892x; 10.4892x over previous
import functools

import jax
import jax.numpy as jnp
from jax import lax
from jax.experimental import pallas as pl
from jax.experimental.pallas import tpu as pltpu
from jax.experimental.pallas import tpu_sc as plsc

N = 10000
E = 320000
D = 128
H = 128

NPAD = 10240
NCORES = 2
NSUB = 16
NW = NCORES * NSUB
CH = 128
EPT = 10112
NITER = EPT // CH
EPAD = NW * EPT
RPT = NPAD // NSUB

_mesh = plsc.VectorSubcoreMesh(core_axis_name="c", subcore_axis_name="s")


@functools.partial(
    pl.kernel,
    out_type=jax.ShapeDtypeStruct((NCORES * NPAD, H), jnp.float32),
    mesh=_mesh,
    scratch_types=[
        pltpu.VMEM((CH,), jnp.int32),
        pltpu.VMEM((CH, H), jnp.float32),
        pltpu.VMEM_SHARED((NPAD, H), jnp.float32),
    ],
)
def _deg_kernel(dst_hbm, out_hbm, idx_d, ones_v, dacc):
    c = lax.axis_index("c")
    s = lax.axis_index("s")
    tile = c * NSUB + s
    r0 = s * RPT
    one16 = jnp.full((16,), 1.0, jnp.float32)

    def fill_ones(i, _):
        for j in range(H // 16):
            ones_v[i, pl.ds(j * 16, 16)] = one16
        return 0

    lax.fori_loop(0, CH, fill_ones, 0)
    for k in range(RPT // CH):
        pltpu.sync_copy(ones_v, dacc.at[pl.ds(r0 + k * CH, CH)])
    plsc.subcore_barrier()

    def body(i, _):
        b = tile * EPT + i * CH
        pltpu.sync_copy(dst_hbm.at[pl.ds(b, CH)], idx_d)
        pltpu.sync_copy(ones_v, dacc.at[idx_d], add=True)
        return 0

    lax.fori_loop(0, NITER, body, 0)
    plsc.subcore_barrier()
    pltpu.sync_copy(dacc.at[pl.ds(r0, RPT)], out_hbm.at[pl.ds(c * NPAD + r0, RPT)])


@functools.partial(
    pl.kernel,
    out_type=jax.ShapeDtypeStruct((NCORES * NPAD, H), jnp.float32),
    mesh=_mesh,
    scratch_types=[
        pltpu.VMEM((CH,), jnp.int32),
        pltpu.VMEM((CH,), jnp.int32),
        pltpu.VMEM((CH, H), jnp.float32),
        pltpu.VMEM_SHARED((NPAD, H), jnp.float32),
        pltpu.SemaphoreType.DMA,
    ],
)
def _scatter_kernel(y_hbm, src_hbm, dst_hbm, out_hbm, idx_s, idx_d, rows, acc, sem):
    c = lax.axis_index("c")
    s = lax.axis_index("s")
    tile = c * NSUB + s
    r0 = s * RPT
    pltpu.sync_copy(y_hbm.at[pl.ds(r0, RPT)], acc.at[pl.ds(r0, RPT)])
    plsc.subcore_barrier()

    def body(i, _):
        b = tile * EPT + i * CH
        pltpu.sync_copy(src_hbm.at[pl.ds(b, CH)], idx_s)
        pltpu.sync_copy(dst_hbm.at[pl.ds(b, CH)], idx_d)
        pltpu.async_copy(y_hbm.at[idx_s], rows, sem).wait()
        pltpu.sync_copy(rows, acc.at[idx_d], add=True)
        return 0

    lax.fori_loop(0, NITER, body, 0)
    plsc.subcore_barrier()
    pltpu.sync_copy(acc.at[pl.ds(r0, RPT)], out_hbm.at[pl.ds(c * NPAD + r0, RPT)])


def _dinv(d0_ref, d1_ref):
    deg = d0_ref[:, 0:1] + d1_ref[:, 0:1] - 1.0
    return lax.rsqrt(deg)


def _mm_scale_body(x_ref, w_ref, d0_ref, d1_ref, o_ref):
    y = jnp.dot(x_ref[...], w_ref[...], preferred_element_type=jnp.float32)
    o_ref[...] = y * _dinv(d0_ref, d1_ref)


def _mid_body(p0_ref, p1_ref, y_ref, d0_ref, d1_ref, b_ref, w_ref, o_ref):
    dinv = _dinv(d0_ref, d1_ref)
    h = (p0_ref[...] + p1_ref[...] - y_ref[...]) * dinv + b_ref[...]
    h = jnp.maximum(h, 0.0)
    o_ref[...] = jnp.dot(h, w_ref[...], preferred_element_type=jnp.float32) * dinv


def _final_body(q0_ref, q1_ref, y_ref, d0_ref, d1_ref, b_ref, o_ref):
    dinv = _dinv(d0_ref, d1_ref)
    o_ref[...] = (q0_ref[...] + q1_ref[...] - y_ref[...]) * dinv + b_ref[...]


_BM = 2048
_row = pl.BlockSpec((_BM, H), lambda i: (i, 0))
_w = pl.BlockSpec((D, H), lambda i: (0, 0))
_b = pl.BlockSpec((1, H), lambda i: (0, 0))
_grid = (NPAD // _BM,)

_mm_scale = pl.pallas_call(
    _mm_scale_body,
    grid=_grid,
    in_specs=[_row, _w, _row, _row],
    out_specs=_row,
    out_shape=jax.ShapeDtypeStruct((NPAD, H), jnp.float32),
)

_mid = pl.pallas_call(
    _mid_body,
    grid=_grid,
    in_specs=[_row, _row, _row, _row, _row, _b, _w],
    out_specs=_row,
    out_shape=jax.ShapeDtypeStruct((NPAD, H), jnp.float32),
)

_final = pl.pallas_call(
    _final_body,
    grid=_grid,
    in_specs=[_row, _row, _row, _row, _row, _b],
    out_specs=_row,
    out_shape=jax.ShapeDtypeStruct((NPAD, H), jnp.float32),
)


def kernel(x, W1, b1, W2, b2, edge_index, scope):
    del scope
    xp = jnp.zeros((NPAD, D), jnp.float32).at[:N].set(x)
    pad = EPAD - E
    srcp = jnp.concatenate([edge_index[0], jnp.zeros((pad,), jnp.int32)])
    dstp = jnp.concatenate([edge_index[1], jnp.full((pad,), N, jnp.int32)])

    degp = _deg_kernel(dstp)
    deg0, deg1 = degp[:NPAD], degp[NPAD:]
    b1r = b1.reshape(1, H)
    b2r = b2.reshape(1, H)

    y1 = _mm_scale(xp, W1, deg0, deg1)
    p = _scatter_kernel(y1, srcp, dstp)
    y2 = _mid(p[:NPAD], p[NPAD:], y1, deg0, deg1, b1r, W2)
    q = _scatter_kernel(y2, srcp, dstp)
    out = _final(q[:NPAD], q[NPAD:], y2, deg0, deg1, b2r)
    return out[:N]
